# baseline (device time: 292789 ns/iter reference)
import jax
import jax.numpy as jnp
from jax import lax
from jax.experimental import pallas as pl
from jax.experimental.pallas import tpu as pltpu

B = 32
H = 16
D = 128
TOK = 32
NB = 256
NZ = 4
PAGES_LOCAL = 256
CHUNK_PAGES = 32
C = PAGES_LOCAL // CHUNK_PAGES
CHUNK_TOK = CHUNK_PAGES * TOK
SCALE = D ** -0.5
NEG_INF = -1e30


def kernel(Q, K, V, bt, lens):
    lens2d = lens.reshape(B, 1)
    Q2 = Q.reshape(B, H * D)
    K2 = K.reshape(PAGES_LOCAL, TOK, H * D)
    V2 = V.reshape(PAGES_LOCAL, TOK, H * D)

    def body(q_ref, k_ref, v_ref, bt_ref, lens_ref, out_ref,
             m_s, l_s, acc_s, count_s, packed_mine, packed_recv,
             send_sems, recv_sems):
        c = pl.program_id(0)
        h = pl.program_id(1)
        my_x = lax.axis_index("x")
        my_y = lax.axis_index("y")
        my_z = lax.axis_index("z")

        @pl.when(c == 0)
        def _():
            m_s[h, :, :] = jnp.full((B, 1), NEG_INF, jnp.float32)
            l_s[h, :, :] = jnp.zeros((B, 1), jnp.float32)
            acc_s[h, :, :] = jnp.zeros((B, D), jnp.float32)

        @pl.when(h == 0)
        def _():
            pids = (my_z * PAGES_LOCAL + c * CHUNK_PAGES
                    + lax.broadcasted_iota(jnp.int32, (1, CHUNK_PAGES, 1), 1))
            btx = bt_ref[:, :].reshape(B, 1, NB)
            j_iota = lax.broadcasted_iota(jnp.int32, (B, 1, NB), 2)
            valid = j_iota < lens_ref[:, :].reshape(B, 1, 1)
            hit = jnp.logical_and(btx == pids, valid).astype(jnp.float32)
            count_chunk = jnp.sum(hit, axis=-1)
            count_s[:, :] = jnp.broadcast_to(
                count_chunk[:, :, None], (B, CHUNK_PAGES, TOK)
            ).reshape(B, CHUNK_TOK)

        qh = q_ref[:, :].astype(jnp.bfloat16)
        kh = k_ref[:, :, :].reshape(CHUNK_TOK, D).astype(jnp.bfloat16)
        vh = v_ref[:, :, :].reshape(CHUNK_TOK, D).astype(jnp.bfloat16)
        s = lax.dot_general(
            qh, kh, (((1,), (1,)), ((), ())),
            preferred_element_type=jnp.float32,
        ) * SCALE
        m_prev = m_s[h, :, :]
        m_new = jnp.maximum(m_prev, jnp.max(s, axis=1, keepdims=True))
        alpha = jnp.exp(m_prev - m_new)
        p = count_s[:, :] * jnp.exp(s - m_new)
        l_s[h, :, :] = l_s[h, :, :] * alpha + jnp.sum(
            p, axis=1, keepdims=True)
        pv = lax.dot_general(
            p.astype(jnp.bfloat16), vh, (((1,), (0,)), ((), ())),
            preferred_element_type=jnp.float32,
        )
        acc_s[h, :, :] = acc_s[h, :, :] * alpha + pv
        m_s[h, :, :] = m_new

        @pl.when(jnp.logical_and(c == C - 1, h == H - 1))
        def _():
            for hh in range(H):
                packed_mine[hh, :, :] = acc_s[hh, :, :]
            m_hb = m_s[:, :, 0]
            l_hb = l_s[:, :, 0]
            packed_mine[H, 0:H, 0:B] = m_hb
            packed_mine[H, H:2 * H, 0:B] = l_hb

            rdmas = []
            for dz in (1, 2, 3):
                rdma = pltpu.make_async_remote_copy(
                    src_ref=packed_mine,
                    dst_ref=packed_recv.at[dz - 1],
                    send_sem=send_sems.at[dz - 1],
                    recv_sem=recv_sems.at[dz - 1],
                    device_id=(my_x, my_y, (my_z + dz) % NZ),
                    device_id_type=pl.DeviceIdType.MESH,
                )
                rdma.start()
                rdmas.append(rdma)
            for rdma in rdmas:
                rdma.wait_recv()

            m_parts = [m_hb] + [
                packed_recv[k, H, 0:H, 0:B] for k in range(NZ - 1)]
            l_parts = [l_hb] + [
                packed_recv[k, H, H:2 * H, 0:B] for k in range(NZ - 1)]
            m_tot = m_parts[0]
            for mp in m_parts[1:]:
                m_tot = jnp.maximum(m_tot, mp)
            scales = [jnp.exp(mp - m_tot) for mp in m_parts]
            l_tot = scales[0] * l_parts[0]
            for sc, lp in zip(scales[1:], l_parts[1:]):
                l_tot = l_tot + sc * lp
            sc_bh = [jnp.transpose(sc) for sc in scales]
            linv_bh = jnp.transpose(1.0 / l_tot)
            for hh in range(H):
                acc_tot = acc_s[hh, :, :] * sc_bh[0][:, hh:hh + 1]
                for k in range(NZ - 1):
                    acc_tot = acc_tot + (
                        packed_recv[k, hh, :, :] * sc_bh[k + 1][:, hh:hh + 1])
                out_ref[:, 0, hh, :] = acc_tot * linv_bh[:, hh:hh + 1]

            for rdma in rdmas:
                rdma.wait_send()

    grid = (C, H)
    return pl.pallas_call(
        body,
        grid=grid,
        out_shape=jax.ShapeDtypeStruct((B, 1, H, D), jnp.float32),
        in_specs=[
            pl.BlockSpec((B, D), lambda c, h: (0, h)),
            pl.BlockSpec((CHUNK_PAGES, TOK, D), lambda c, h: (c, 0, h)),
            pl.BlockSpec((CHUNK_PAGES, TOK, D), lambda c, h: (c, 0, h)),
            pl.BlockSpec((B, NB), lambda c, h: (0, 0)),
            pl.BlockSpec((B, 1), lambda c, h: (0, 0)),
        ],
        out_specs=pl.BlockSpec((B, 1, H, D), lambda c, h: (0, 0, 0, 0)),
        scratch_shapes=[
            pltpu.VMEM((H, B, 1), jnp.float32),
            pltpu.VMEM((H, B, 1), jnp.float32),
            pltpu.VMEM((H, B, D), jnp.float32),
            pltpu.VMEM((B, CHUNK_TOK), jnp.float32),
            pltpu.VMEM((H + 1, B, D), jnp.float32),
            pltpu.VMEM((NZ - 1, H + 1, B, D), jnp.float32),
            pltpu.SemaphoreType.DMA((NZ - 1,)),
            pltpu.SemaphoreType.DMA((NZ - 1,)),
        ],
        compiler_params=pltpu.CompilerParams(
            dimension_semantics=("arbitrary", "arbitrary"),
            vmem_limit_bytes=100 * 1024 * 1024,
        ),
    )(Q2, K2, V2, bt, lens2d)


# device time: 80397 ns/iter; 3.6418x vs baseline; 3.6418x over previous
import jax
import jax.numpy as jnp
from jax import lax
from jax.experimental import pallas as pl
from jax.experimental.pallas import tpu as pltpu

B = 32
H = 16
D = 128
TOK = 32
NB = 256
NZ = 4
PAGES_LOCAL = 256
MY_PAGES = 64
CHUNK_PAGES = 32
C = MY_PAGES // CHUNK_PAGES
CHUNK_TOK = CHUNK_PAGES * TOK
SCALE = D ** -0.5
NEG_INF = -1e30


def kernel(Q, K, V, bt, lens):
    lens2d = lens.reshape(B, 1)

    def body(q_ref, k_hbm, v_hbm, bt_ref, lens_ref, out_ref,
             k_buf, v_buf, pk1_mine, pk1_recv, pk2_mine, pk2_recv,
             k_sems, v_sems, send1, recv1, send2, recv2):
        my_x = lax.axis_index("x")
        my_y = lax.axis_index("y")
        my_z = lax.axis_index("z")
        q_xy = my_x * 2 + my_y
        page0 = q_xy * MY_PAGES

        copies = []
        for c in range(C):
            kc = pltpu.make_async_copy(
                k_hbm.at[pl.ds(page0 + c * CHUNK_PAGES, CHUNK_PAGES)],
                k_buf.at[c], k_sems.at[c])
            vc = pltpu.make_async_copy(
                v_hbm.at[pl.ds(page0 + c * CHUNK_PAGES, CHUNK_PAGES)],
                v_buf.at[c], v_sems.at[c])
            kc.start()
            vc.start()
            copies.append((kc, vc))

        m_st = [jnp.full((B, 1), NEG_INF, jnp.float32) for _ in range(H)]
        l_st = [jnp.zeros((B, 1), jnp.float32) for _ in range(H)]
        a_st = [jnp.zeros((B, D), jnp.float32) for _ in range(H)]

        j_iota = lax.broadcasted_iota(jnp.int32, (B, 1, NB), 2)
        valid = j_iota < lens_ref[:, :].reshape(B, 1, 1)
        btx = bt_ref[:, :].reshape(B, 1, NB)

        for c in range(C):
            copies[c][0].wait()
            copies[c][1].wait()

            pids = (my_z * PAGES_LOCAL + page0 + c * CHUNK_PAGES
                    + lax.broadcasted_iota(jnp.int32, (1, CHUNK_PAGES, 1), 1))
            hit = jnp.logical_and(btx == pids, valid).astype(jnp.float32)
            count_chunk = jnp.sum(hit, axis=-1)
            count_tok = jnp.broadcast_to(
                count_chunk[:, :, None], (B, CHUNK_PAGES, TOK)
            ).reshape(B, CHUNK_TOK)

            for h in range(H):
                qh = q_ref[:, 0, h, :].astype(jnp.bfloat16)
                kh = k_buf[c, :, :, h, :].reshape(
                    CHUNK_TOK, D).astype(jnp.bfloat16)
                vh = v_buf[c, :, :, h, :].reshape(
                    CHUNK_TOK, D).astype(jnp.bfloat16)
                s = lax.dot_general(
                    qh, kh, (((1,), (1,)), ((), ())),
                    preferred_element_type=jnp.float32,
                ) * SCALE
                m_new = jnp.maximum(m_st[h], jnp.max(s, 1, keepdims=True))
                alpha = jnp.exp(m_st[h] - m_new)
                p = count_tok * jnp.exp(s - m_new)
                l_st[h] = l_st[h] * alpha + jnp.sum(p, 1, keepdims=True)
                pv = lax.dot_general(
                    p.astype(jnp.bfloat16), vh, (((1,), (0,)), ((), ())),
                    preferred_element_type=jnp.float32,
                )
                a_st[h] = a_st[h] * alpha + pv
                m_st[h] = m_new

        def pack(dst, accs, m_hb, l_hb):
            for hh in range(H):
                dst[hh, :, :] = accs[hh]
            dst[H, 0:H, 0:B] = m_hb
            dst[H, H:2 * H, 0:B] = l_hb

        def exchange(pk_mine, pk_recv, send_sems, recv_sems, peer_fn):
            rdmas = []
            for d in (1, 2, 3):
                rdma = pltpu.make_async_remote_copy(
                    src_ref=pk_mine,
                    dst_ref=pk_recv.at[d - 1],
                    send_sem=send_sems.at[d - 1],
                    recv_sem=recv_sems.at[d - 1],
                    device_id=peer_fn(d),
                    device_id_type=pl.DeviceIdType.MESH,
                )
                rdma.start()
                rdmas.append(rdma)
            for rdma in rdmas:
                rdma.wait_recv()
            return rdmas

        def merge(pk_recv, accs, m_hb, l_hb):
            m_parts = [m_hb] + [
                pk_recv[k, H, 0:H, 0:B] for k in range(3)]
            l_parts = [l_hb] + [
                pk_recv[k, H, H:2 * H, 0:B] for k in range(3)]
            m_tot = m_parts[0]
            for mp in m_parts[1:]:
                m_tot = jnp.maximum(m_tot, mp)
            scales = [jnp.exp(mp - m_tot) for mp in m_parts]
            l_tot = scales[0] * l_parts[0]
            for sc, lp in zip(scales[1:], l_parts[1:]):
                l_tot = l_tot + sc * lp
            sc_bh = [jnp.transpose(sc) for sc in scales]
            accs_out = []
            for hh in range(H):
                acc_tot = accs[hh] * sc_bh[0][:, hh:hh + 1]
                for k in range(3):
                    acc_tot = acc_tot + (
                        pk_recv[k, hh, :, :] * sc_bh[k + 1][:, hh:hh + 1])
                accs_out.append(acc_tot)
            return accs_out, m_tot, l_tot

        m_hb = jnp.transpose(jnp.concatenate(m_st, axis=1))
        l_hb = jnp.transpose(jnp.concatenate(l_st, axis=1))

        pack(pk1_mine, a_st, m_hb, l_hb)
        r1 = exchange(
            pk1_mine, pk1_recv, send1, recv1,
            lambda d: ((q_xy ^ d) // 2, (q_xy ^ d) % 2, my_z))
        a1, m1, l1 = merge(pk1_recv, a_st, m_hb, l_hb)

        pack(pk2_mine, a1, m1, l1)
        r2 = exchange(
            pk2_mine, pk2_recv, send2, recv2,
            lambda d: (my_x, my_y, (my_z + d) % NZ))
        a2, _, l2 = merge(pk2_recv, a1, m1, l1)

        linv_bh = jnp.transpose(1.0 / l2)
        for hh in range(H):
            out_ref[:, 0, hh, :] = a2[hh] * linv_bh[:, hh:hh + 1]

        for rdma in r1 + r2:
            rdma.wait_send()

    return pl.pallas_call(
        body,
        out_shape=jax.ShapeDtypeStruct((B, 1, H, D), jnp.float32),
        in_specs=[
            pl.BlockSpec(memory_space=pltpu.VMEM),
            pl.BlockSpec(memory_space=pl.ANY),
            pl.BlockSpec(memory_space=pl.ANY),
            pl.BlockSpec(memory_space=pltpu.VMEM),
            pl.BlockSpec(memory_space=pltpu.VMEM),
        ],
        out_specs=pl.BlockSpec(memory_space=pltpu.VMEM),
        scratch_shapes=[
            pltpu.VMEM((C, CHUNK_PAGES, TOK, H, D), jnp.float32),
            pltpu.VMEM((C, CHUNK_PAGES, TOK, H, D), jnp.float32),
            pltpu.VMEM((H + 1, B, D), jnp.float32),
            pltpu.VMEM((3, H + 1, B, D), jnp.float32),
            pltpu.VMEM((H + 1, B, D), jnp.float32),
            pltpu.VMEM((3, H + 1, B, D), jnp.float32),
            pltpu.SemaphoreType.DMA((C,)),
            pltpu.SemaphoreType.DMA((C,)),
            pltpu.SemaphoreType.DMA((3,)),
            pltpu.SemaphoreType.DMA((3,)),
            pltpu.SemaphoreType.DMA((3,)),
            pltpu.SemaphoreType.DMA((3,)),
        ],
        compiler_params=pltpu.CompilerParams(
            vmem_limit_bytes=100 * 1024 * 1024,
        ),
    )(Q, K, V, bt, lens2d)


# device time: 79735 ns/iter; 3.6720x vs baseline; 1.0083x over previous
import jax
import jax.numpy as jnp
from jax import lax
from jax.experimental import pallas as pl
from jax.experimental.pallas import tpu as pltpu

B = 32
H = 16
HG = 2
HPG = H // HG
D = 128
TOK = 32
NB = 256
NZ = 4
PAGES_LOCAL = 256
MY_PAGES = 64
MY_TOK = MY_PAGES * TOK
SCALE = D ** -0.5


def kernel(Q, K, V, bt, lens):
    lens2d = lens.reshape(B, 1)

    def body(q_ref, k_hbm, v_hbm, bt_ref, lens_ref, out_ref,
             k_buf, v_buf, pk1_mine, pk1_recv, pk2_mine, pk2_recv,
             kv_sems, send1, recv1, send2, recv2):
        my_x = lax.axis_index("x")
        my_y = lax.axis_index("y")
        my_z = lax.axis_index("z")
        q_xy = my_x * 2 + my_y
        page0 = q_xy * MY_PAGES

        kc = pltpu.make_async_copy(
            k_hbm.at[pl.ds(page0, MY_PAGES)], k_buf, kv_sems.at[0])
        vc = pltpu.make_async_copy(
            v_hbm.at[pl.ds(page0, MY_PAGES)], v_buf, kv_sems.at[1])
        kc.start()
        vc.start()

        pids = (my_z * PAGES_LOCAL + page0
                + lax.broadcasted_iota(jnp.int32, (1, MY_PAGES, 1), 1))
        j_iota = lax.broadcasted_iota(jnp.int32, (B, 1, NB), 2)
        valid = j_iota < lens_ref[:, :].reshape(B, 1, 1)
        btx = bt_ref[:, :].reshape(B, 1, NB)
        hit = jnp.logical_and(btx == pids, valid).astype(jnp.float32)
        count_page = jnp.sum(hit, axis=-1)
        count_tok = jnp.broadcast_to(
            count_page[:, :, None], (B, MY_PAGES, TOK)
        ).reshape(B, MY_TOK)

        kc.wait()
        vc.wait()

        def peer1(d):
            return ((q_xy ^ d) // 2, (q_xy ^ d) % 2, my_z)

        def peer2(d):
            return (my_x, my_y, (my_z + d) % NZ)

        def exchange(pk_mine, pk_recv, send_sems, recv_sems, peer_fn, g):
            rdmas = []
            for d in (1, 2, 3):
                rdma = pltpu.make_async_remote_copy(
                    src_ref=pk_mine.at[g],
                    dst_ref=pk_recv.at[g, d - 1],
                    send_sem=send_sems.at[g, d - 1],
                    recv_sem=recv_sems.at[g, d - 1],
                    device_id=peer_fn(d),
                    device_id_type=pl.DeviceIdType.MESH,
                )
                rdma.start()
                rdmas.append(rdma)
            return rdmas

        def pack(dst_ref, g, accs, m_hb, l_hb):
            for i in range(HPG):
                dst_ref[g, i, :, :] = accs[i]
            dst_ref[g, HPG, 0:HPG, 0:B] = m_hb
            dst_ref[g, HPG, HPG:2 * HPG, 0:B] = l_hb

        def merge(pk_recv, g, accs, m_hb, l_hb):
            m_parts = [m_hb] + [
                pk_recv[g, k, HPG, 0:HPG, 0:B] for k in range(3)]
            l_parts = [l_hb] + [
                pk_recv[g, k, HPG, HPG:2 * HPG, 0:B] for k in range(3)]
            m_tot = m_parts[0]
            for mp in m_parts[1:]:
                m_tot = jnp.maximum(m_tot, mp)
            scales = [jnp.exp(mp - m_tot) for mp in m_parts]
            l_tot = scales[0] * l_parts[0]
            for sc, lp in zip(scales[1:], l_parts[1:]):
                l_tot = l_tot + sc * lp
            sc_bh = [jnp.transpose(sc) for sc in scales]
            accs_out = []
            for i in range(HPG):
                acc_tot = accs[i] * sc_bh[0][:, i:i + 1]
                for k in range(3):
                    acc_tot = acc_tot + (
                        pk_recv[g, k, i, :, :] * sc_bh[k + 1][:, i:i + 1])
                accs_out.append(acc_tot)
            return accs_out, m_tot, l_tot

        part = []
        r1 = []
        for g in range(HG):
            accs, ms, ls = [], [], []
            for i in range(HPG):
                h = g * HPG + i
                qh = q_ref[:, 0, h, :].astype(jnp.bfloat16)
                kh = k_buf[:, :, h, :].reshape(
                    MY_TOK, D).astype(jnp.bfloat16)
                vh = v_buf[:, :, h, :].reshape(
                    MY_TOK, D).astype(jnp.bfloat16)
                s = lax.dot_general(
                    qh, kh, (((1,), (1,)), ((), ())),
                    preferred_element_type=jnp.float32,
                ) * SCALE
                m = jnp.max(s, 1, keepdims=True)
                p = count_tok * jnp.exp(s - m)
                l = jnp.sum(p, 1, keepdims=True)
                acc = lax.dot_general(
                    p.astype(jnp.bfloat16), vh, (((1,), (0,)), ((), ())),
                    preferred_element_type=jnp.float32,
                )
                accs.append(acc)
                ms.append(m)
                ls.append(l)
            m_hb = jnp.transpose(jnp.concatenate(ms, axis=1))
            l_hb = jnp.transpose(jnp.concatenate(ls, axis=1))
            pack(pk1_mine, g, accs, m_hb, l_hb)
            r1.append(exchange(pk1_mine, pk1_recv, send1, recv1, peer1, g))
            part.append((accs, m_hb, l_hb))

        r2 = []
        zpart = []
        for g in range(HG):
            for rdma in r1[g]:
                rdma.wait_recv()
            a1, m1, l1 = merge(pk1_recv, g, *part[g])
            pack(pk2_mine, g, a1, m1, l1)
            r2.append(exchange(pk2_mine, pk2_recv, send2, recv2, peer2, g))
            zpart.append((a1, m1, l1))

        for g in range(HG):
            for rdma in r2[g]:
                rdma.wait_recv()
            a2, _, l2 = merge(pk2_recv, g, *zpart[g])
            linv_bh = jnp.transpose(1.0 / l2)
            for i in range(HPG):
                out_ref[:, 0, g * HPG + i, :] = a2[i] * linv_bh[:, i:i + 1]

        for rdmas in r1 + r2:
            for rdma in rdmas:
                rdma.wait_send()

    return pl.pallas_call(
        body,
        out_shape=jax.ShapeDtypeStruct((B, 1, H, D), jnp.float32),
        in_specs=[
            pl.BlockSpec(memory_space=pltpu.MemorySpace.VMEM),
            pl.BlockSpec(memory_space=pl.ANY),
            pl.BlockSpec(memory_space=pl.ANY),
            pl.BlockSpec(memory_space=pltpu.MemorySpace.VMEM),
            pl.BlockSpec(memory_space=pltpu.MemorySpace.VMEM),
        ],
        out_specs=pl.BlockSpec(memory_space=pltpu.MemorySpace.VMEM),
        scratch_shapes=[
            pltpu.VMEM((MY_PAGES, TOK, H, D), jnp.float32),
            pltpu.VMEM((MY_PAGES, TOK, H, D), jnp.float32),
            pltpu.VMEM((HG, HPG + 1, B, D), jnp.float32),
            pltpu.VMEM((HG, 3, HPG + 1, B, D), jnp.float32),
            pltpu.VMEM((HG, HPG + 1, B, D), jnp.float32),
            pltpu.VMEM((HG, 3, HPG + 1, B, D), jnp.float32),
            pltpu.SemaphoreType.DMA((2,)),
            pltpu.SemaphoreType.DMA((HG, 3)),
            pltpu.SemaphoreType.DMA((HG, 3)),
            pltpu.SemaphoreType.DMA((HG, 3)),
            pltpu.SemaphoreType.DMA((HG, 3)),
        ],
        compiler_params=pltpu.CompilerParams(
            vmem_limit_bytes=100 * 1024 * 1024,
        ),
    )(Q, K, V, bt, lens2d)


# device time: 42247 ns/iter; 6.9304x vs baseline; 1.8874x over previous
import jax
import jax.numpy as jnp
from jax import lax
from jax.experimental import pallas as pl
from jax.experimental.pallas import tpu as pltpu

B = 32
H = 16
HG = 2
HPG = H // HG
D = 128
TOK = 32
NB = 256
NZ = 4
PAGES_LOCAL = 256
MY_PAGES = 64
MY_TOK = MY_PAGES * TOK
SCALE = D ** -0.5


def kernel(Q, K, V, bt, lens):
    lens2d = lens.reshape(B, 1)

    def body(q_ref, k_hbm, v_hbm, bt_ref, lens_ref, out_ref,
             kh_buf, vh_buf,
             pk1_mine, pk1_recv, pk2_mine, pk2_recv,
             kh_sems, vh_sems, send1, recv1, send2, recv2):
        my_x = lax.axis_index("x")
        my_y = lax.axis_index("y")
        my_z = lax.axis_index("z")
        q_xy = my_x * 2 + my_y
        page0 = q_xy * MY_PAGES

        reorg = []
        for h in range(H):
            kr = pltpu.make_async_copy(
                k_hbm.at[pl.ds(page0, MY_PAGES), :, h, :],
                kh_buf.at[h], kh_sems.at[h])
            vr = pltpu.make_async_copy(
                v_hbm.at[pl.ds(page0, MY_PAGES), :, h, :],
                vh_buf.at[h], vh_sems.at[h])
            kr.start()
            vr.start()
            reorg.append((kr, vr))

        pids = (my_z * PAGES_LOCAL + page0
                + lax.broadcasted_iota(jnp.int32, (1, MY_PAGES, 1), 1))
        j_iota = lax.broadcasted_iota(jnp.int32, (B, 1, NB), 2)
        valid = j_iota < lens_ref[:, :].reshape(B, 1, 1)
        btx = bt_ref[:, :].reshape(B, 1, NB)
        hit = jnp.logical_and(btx == pids, valid).astype(jnp.float32)
        count_page = jnp.sum(hit, axis=-1)
        count_tok = jnp.broadcast_to(
            count_page[:, :, None], (B, MY_PAGES, TOK)
        ).reshape(B, MY_TOK)

        def peer1(d):
            return ((q_xy ^ d) // 2, (q_xy ^ d) % 2, my_z)

        def peer2(d):
            return (my_x, my_y, (my_z + d) % NZ)

        def exchange(pk_mine, pk_recv, send_sems, recv_sems, peer_fn, g):
            rdmas = []
            for d in (1, 2, 3):
                rdma = pltpu.make_async_remote_copy(
                    src_ref=pk_mine.at[g],
                    dst_ref=pk_recv.at[g, d - 1],
                    send_sem=send_sems.at[g, d - 1],
                    recv_sem=recv_sems.at[g, d - 1],
                    device_id=peer_fn(d),
                    device_id_type=pl.DeviceIdType.MESH,
                )
                rdma.start()
                rdmas.append(rdma)
            return rdmas

        def pack(dst_ref, g, accs, m_hb, l_hb):
            for i in range(HPG):
                dst_ref[g, i, :, :] = accs[i]
            dst_ref[g, HPG, 0:HPG, 0:B] = m_hb
            dst_ref[g, HPG, HPG:2 * HPG, 0:B] = l_hb

        def merge(pk_recv, g, accs, m_hb, l_hb):
            m_parts = [m_hb] + [
                pk_recv[g, k, HPG, 0:HPG, 0:B] for k in range(3)]
            l_parts = [l_hb] + [
                pk_recv[g, k, HPG, HPG:2 * HPG, 0:B] for k in range(3)]
            m_tot = m_parts[0]
            for mp in m_parts[1:]:
                m_tot = jnp.maximum(m_tot, mp)
            scales = [jnp.exp(mp - m_tot) for mp in m_parts]
            l_tot = scales[0] * l_parts[0]
            for sc, lp in zip(scales[1:], l_parts[1:]):
                l_tot = l_tot + sc * lp
            sc_bh = [jnp.transpose(sc) for sc in scales]
            accs_out = []
            for i in range(HPG):
                acc_tot = accs[i] * sc_bh[0][:, i:i + 1]
                for k in range(3):
                    acc_tot = acc_tot + (
                        pk_recv[g, k, i, :, :] * sc_bh[k + 1][:, i:i + 1])
                accs_out.append(acc_tot)
            return accs_out, m_tot, l_tot

        part = []
        r1 = []
        for g in range(HG):
            accs, ms, ls = [], [], []
            for i in range(HPG):
                h = g * HPG + i
                qh = q_ref[:, 0, h, :].astype(jnp.bfloat16)
                reorg[h][0].wait()
                reorg[h][1].wait()
                kh = kh_buf[h].reshape(MY_TOK, D).astype(jnp.bfloat16)
                vh = vh_buf[h].reshape(MY_TOK, D).astype(jnp.bfloat16)
                s = lax.dot_general(
                    qh, kh, (((1,), (1,)), ((), ())),
                    preferred_element_type=jnp.float32,
                ) * SCALE
                m = jnp.max(s, 1, keepdims=True)
                p = count_tok * jnp.exp(s - m)
                l = jnp.sum(p, 1, keepdims=True)
                acc = lax.dot_general(
                    p.astype(jnp.bfloat16), vh, (((1,), (0,)), ((), ())),
                    preferred_element_type=jnp.float32,
                )
                accs.append(acc)
                ms.append(m)
                ls.append(l)
            m_hb = jnp.transpose(jnp.concatenate(ms, axis=1))
            l_hb = jnp.transpose(jnp.concatenate(ls, axis=1))
            pack(pk1_mine, g, accs, m_hb, l_hb)
            r1.append(exchange(pk1_mine, pk1_recv, send1, recv1, peer1, g))
            part.append((accs, m_hb, l_hb))

        r2 = []
        zpart = []
        for g in range(HG):
            for rdma in r1[g]:
                rdma.wait_recv()
            a1, m1, l1 = merge(pk1_recv, g, *part[g])
            pack(pk2_mine, g, a1, m1, l1)
            r2.append(exchange(pk2_mine, pk2_recv, send2, recv2, peer2, g))
            zpart.append((a1, m1, l1))

        for g in range(HG):
            for rdma in r2[g]:
                rdma.wait_recv()
            a2, _, l2 = merge(pk2_recv, g, *zpart[g])
            linv_bh = jnp.transpose(1.0 / l2)
            for i in range(HPG):
                out_ref[:, 0, g * HPG + i, :] = a2[i] * linv_bh[:, i:i + 1]

        for rdmas in r1 + r2:
            for rdma in rdmas:
                rdma.wait_send()

    return pl.pallas_call(
        body,
        out_shape=jax.ShapeDtypeStruct((B, 1, H, D), jnp.float32),
        in_specs=[
            pl.BlockSpec(memory_space=pltpu.MemorySpace.VMEM),
            pl.BlockSpec(memory_space=pl.ANY),
            pl.BlockSpec(memory_space=pl.ANY),
            pl.BlockSpec(memory_space=pltpu.MemorySpace.VMEM),
            pl.BlockSpec(memory_space=pltpu.MemorySpace.VMEM),
        ],
        out_specs=pl.BlockSpec(memory_space=pltpu.MemorySpace.VMEM),
        scratch_shapes=[
            pltpu.VMEM((H, MY_PAGES, TOK, D), jnp.float32),
            pltpu.VMEM((H, MY_PAGES, TOK, D), jnp.float32),
            pltpu.VMEM((HG, HPG + 1, B, D), jnp.float32),
            pltpu.VMEM((HG, 3, HPG + 1, B, D), jnp.float32),
            pltpu.VMEM((HG, HPG + 1, B, D), jnp.float32),
            pltpu.VMEM((HG, 3, HPG + 1, B, D), jnp.float32),
            pltpu.SemaphoreType.DMA((H,)),
            pltpu.SemaphoreType.DMA((H,)),
            pltpu.SemaphoreType.DMA((HG, 3)),
            pltpu.SemaphoreType.DMA((HG, 3)),
            pltpu.SemaphoreType.DMA((HG, 3)),
            pltpu.SemaphoreType.DMA((HG, 3)),
        ],
        compiler_params=pltpu.CompilerParams(
            vmem_limit_bytes=100 * 1024 * 1024,
        ),
    )(Q, K, V, bt, lens2d)
